# baseline (speedup 1.0000x reference)
import functools

import jax
import jax.numpy as jnp
from jax.experimental import pallas as pl
from jax.experimental.pallas import tpu as pltpu

TS = 512
NBUF = 4


def _body(idx_ref, x_hbm, dw_hbm, db_hbm, uw_hbm, o_hbm,
          x_buf, o_buf, dwv, dbv, uwv, sem_x, sem_o, sem_w,
          *, B, S, C, D, SB, T):
    t = pl.program_id(0)
    b = t // SB
    s = t % SB
    slot = t % NBUF

    @pl.when(t == 0)
    def _prologue():
        for bb in range(B):
            e = idx_ref[bb]
            pltpu.make_async_copy(dw_hbm.at[0, e], dwv.at[bb], sem_w).start()
            pltpu.make_async_copy(db_hbm.at[0, e], dbv.at[bb], sem_w).start()
            pltpu.make_async_copy(uw_hbm.at[0, e], uwv.at[bb], sem_w).start()
        for k in range(min(NBUF, T)):
            bk, sk = k // SB, k % SB
            pltpu.make_async_copy(
                x_hbm.at[bk, pl.ds(sk * TS, TS), :], x_buf.at[k], sem_x.at[k]
            ).start()
        for bb in range(B):
            e = idx_ref[bb]
            pltpu.make_async_copy(dw_hbm.at[0, e], dwv.at[bb], sem_w).wait()
            pltpu.make_async_copy(db_hbm.at[0, e], dbv.at[bb], sem_w).wait()
            pltpu.make_async_copy(uw_hbm.at[0, e], uwv.at[bb], sem_w).wait()

    pltpu.make_async_copy(
        x_hbm.at[b, pl.ds(s * TS, TS), :], x_buf.at[slot], sem_x.at[slot]
    ).wait()

    @pl.when(t >= NBUF)
    def _wait_out_slot():
        tp = t - NBUF
        bp = tp // SB
        sp = tp % SB
        pltpu.make_async_copy(
            o_buf.at[slot], o_hbm.at[0, bp, pl.ds(sp * TS, TS), :],
            sem_o.at[slot],
        ).wait()

    xb = x_buf[slot].astype(jnp.bfloat16)
    dw = dwv[b].astype(jnp.bfloat16)
    z = jnp.dot(xb, dw, preferred_element_type=jnp.float32) + dbv[b][None, :]
    z = z * jax.nn.sigmoid(z)

    tn = t + NBUF
    bn = tn // SB
    sn = tn % SB
    for k in range(NBUF):

        @pl.when(jnp.logical_and(tn < T, slot == k))
        def _issue_next_x(k=k):
            pltpu.make_async_copy(
                x_hbm.at[bn, pl.ds(sn * TS, TS), :], x_buf.at[k], sem_x.at[k]
            ).start()

    o_buf[slot] = jnp.dot(
        z.astype(jnp.bfloat16),
        uwv[b].astype(jnp.bfloat16),
        preferred_element_type=jnp.float32,
    )

    for k in range(NBUF):

        @pl.when(slot == k)
        def _issue_out(k=k):
            pltpu.make_async_copy(
                o_buf.at[k], o_hbm.at[0, b, pl.ds(s * TS, TS), :], sem_o.at[k]
            ).start()

    @pl.when(t == T - 1)
    def _drain():
        for tq in range(max(0, T - NBUF), T):
            bq, sq = tq // SB, tq % SB
            pltpu.make_async_copy(
                o_buf.at[tq % NBUF],
                o_hbm.at[0, bq, pl.ds(sq * TS, TS), :],
                sem_o.at[tq % NBUF],
            ).wait()


@jax.jit
def kernel(x, expert_index, down_w, down_b, up_w):
    B, S, C = x.shape
    M, N, _, D = down_w.shape
    SB = S // TS
    T = M * B * SB

    idx = expert_index.astype(jnp.int32).reshape(M * B)

    grid_spec = pltpu.PrefetchScalarGridSpec(
        num_scalar_prefetch=1,
        grid=(T,),
        in_specs=[
            pl.BlockSpec(memory_space=pltpu.MemorySpace.HBM),
            pl.BlockSpec(memory_space=pltpu.MemorySpace.HBM),
            pl.BlockSpec(memory_space=pltpu.MemorySpace.HBM),
            pl.BlockSpec(memory_space=pltpu.MemorySpace.HBM),
        ],
        out_specs=pl.BlockSpec(memory_space=pltpu.MemorySpace.HBM),
        scratch_shapes=[
            pltpu.VMEM((NBUF, TS, C), jnp.float32),
            pltpu.VMEM((NBUF, TS, C), jnp.float32),
            pltpu.VMEM((B, C, D), jnp.float32),
            pltpu.VMEM((B, D), jnp.float32),
            pltpu.VMEM((B, D, C), jnp.float32),
            pltpu.SemaphoreType.DMA((NBUF,)),
            pltpu.SemaphoreType.DMA((NBUF,)),
            pltpu.SemaphoreType.DMA,
        ],
    )

    out = pl.pallas_call(
        functools.partial(_body, B=B, S=S, C=C, D=D, SB=SB, T=T),
        grid_spec=grid_spec,
        out_shape=jax.ShapeDtypeStruct((M, B, S, C), jnp.float32),
        compiler_params=pltpu.CompilerParams(
            dimension_semantics=("arbitrary",),
        ),
    )(idx, x, down_w, down_b, up_w)
    return out


# speedup vs baseline: 1.4121x; 1.4121x over previous
import jax
import jax.numpy as jnp
from jax.experimental import pallas as pl
from jax.experimental.pallas import tpu as pltpu

TS = 1024


def _adapter_body(xl_ref, xh_ref, dwl_ref, dwh_ref, db_ref, uw_ref, o_ref):
    xl = xl_ref[0]
    xh = xh_ref[0]
    dwl = dwl_ref[0, 0]
    dwh = dwh_ref[0, 0]
    db = db_ref[0, 0, 0]
    uw = uw_ref[0, 0]
    z = (
        jnp.dot(xl, dwl, preferred_element_type=jnp.float32)
        + jnp.dot(xh, dwh, preferred_element_type=jnp.float32)
        + db[None, :]
    )
    z = z * jax.nn.sigmoid(z)
    o_ref[0, 0] = jnp.dot(z, uw, preferred_element_type=jnp.float32)


@jax.jit
def kernel(x, expert_index, down_w, down_b, up_w):
    B, S, C = x.shape
    M, N, _, D = down_w.shape
    CH = C // 2
    s_blocks = S // TS

    idx = expert_index.astype(jnp.int32)
    m = jnp.arange(M)[:, None]
    bdw = down_w[m, idx]
    bdb = down_b[m, idx].reshape(M, B, 1, D)
    buw = up_w[m, idx]

    grid = (M, B, s_blocks)

    out = pl.pallas_call(
        _adapter_body,
        grid=grid,
        in_specs=[
            pl.BlockSpec((1, TS, CH), lambda mm, b, s: (b, s, 0)),
            pl.BlockSpec((1, TS, CH), lambda mm, b, s: (b, s, 1)),
            pl.BlockSpec((1, 1, CH, D), lambda mm, b, s: (mm, b, 0, 0)),
            pl.BlockSpec((1, 1, CH, D), lambda mm, b, s: (mm, b, 1, 0)),
            pl.BlockSpec((1, 1, 1, D), lambda mm, b, s: (mm, b, 0, 0)),
            pl.BlockSpec((1, 1, D, C), lambda mm, b, s: (mm, b, 0, 0)),
        ],
        out_specs=pl.BlockSpec((1, 1, TS, C), lambda mm, b, s: (mm, b, s, 0)),
        out_shape=jax.ShapeDtypeStruct((M, B, S, C), jnp.float32),
        compiler_params=pltpu.CompilerParams(
            dimension_semantics=("parallel", "parallel", "parallel"),
        ),
    )(x, x, bdw, bdw, bdb, buw)
    return out
